# baseline (speedup 1.0000x reference)
import functools

import jax
import jax.numpy as jnp
from jax import lax
from jax.experimental import pallas as pl
from jax.experimental.pallas import tpu as pltpu
from jax.experimental.pallas import tpu_sc as plsc

BATCH = 4096
SEQ = 50
EMBED_DIM = 128

NUM_CORES = 2
NUM_SUBCORES = 16
NUM_WORKERS = NUM_CORES * NUM_SUBCORES

BATCH_PER_WORKER = BATCH // NUM_WORKERS
ROWS_PER_WORKER = BATCH_PER_WORKER * SEQ
CB = 4
CHUNK = CB * SEQ
NUM_CHUNKS = BATCH_PER_WORKER // CB
NBUF = 4

_mesh = plsc.VectorSubcoreMesh(
    core_axis_name="c", subcore_axis_name="s",
    num_cores=NUM_CORES, num_subcores=NUM_SUBCORES,
)


@functools.partial(
    pl.kernel,
    out_type=jax.ShapeDtypeStruct((BATCH, SEQ, EMBED_DIM), jnp.float32),
    mesh=_mesh,
    compiler_params=pltpu.CompilerParams(use_tc_tiling_on_sc=True),
    scratch_types=[
        pltpu.VMEM((ROWS_PER_WORKER,), jnp.int32),
        [pltpu.VMEM((CHUNK, EMBED_DIM), jnp.float32) for _ in range(NBUF)],
        [pltpu.SemaphoreType.DMA for _ in range(NBUF)],
        [pltpu.SemaphoreType.DMA for _ in range(NBUF)],
    ],
)
def _embedding_gather(idx_hbm, table_hbm, out_hbm, idx_v, bufs, gsems, osems):
    wid = lax.axis_index("s") * NUM_CORES + lax.axis_index("c")
    batch0 = wid * BATCH_PER_WORKER
    pltpu.sync_copy(idx_hbm.at[wid], idx_v)

    def start_gather(j, b):
        chunk_idx = idx_v.at[pl.ds(j * CHUNK, CHUNK)]
        pltpu.async_copy(table_hbm.at[chunk_idx], bufs[b], gsems[b])

    def start_out(j, b):
        for i in range(CB):
            pltpu.async_copy(bufs[b].at[pl.ds(i * SEQ, SEQ)],
                             out_hbm.at[batch0 + j * CB + i], osems[b])

    def wait_out(b):
        for i in range(CB):
            pltpu.make_async_copy(bufs[b].at[pl.ds(i * SEQ, SEQ)],
                                  out_hbm.at[0], osems[b]).wait()

    start_gather(0, 0)
    start_gather(1, 1)

    @pl.loop(0, NUM_CHUNKS, step=NBUF)
    def _outer(j0):
        for b in range(NBUF):
            j = j0 + b
            bn = (b + 2) % NBUF
            pltpu.make_async_copy(table_hbm.at[idx_v.at[pl.ds(0, CHUNK)]],
                                  bufs[b], gsems[b]).wait()
            start_out(j, b)

            @pl.when(j >= 2)
            def _wait_prev_out():
                wait_out(bn)

            @pl.when(j + 2 < NUM_CHUNKS)
            def _next_gather():
                start_gather(j + 2, bn)

    for b in ((NUM_CHUNKS - 2) % NBUF, (NUM_CHUNKS - 1) % NBUF):
        wait_out(b)


def kernel(news, table):
    idx = news.reshape(NUM_WORKERS, ROWS_PER_WORKER)
    return _embedding_gather(idx, table)


# speedup vs baseline: 1.7837x; 1.7837x over previous
import functools

import jax
import jax.numpy as jnp
from jax import lax
from jax.experimental import pallas as pl
from jax.experimental.pallas import tpu as pltpu
from jax.experimental.pallas import tpu_sc as plsc

BATCH = 4096
SEQ = 50
EMBED_DIM = 128

NUM_CORES = 2
NUM_SUBCORES = 16
NUM_WORKERS = NUM_CORES * NUM_SUBCORES

TOTAL = BATCH * SEQ
ROWS_PER_WORKER = TOTAL // NUM_WORKERS
CHUNK = 80
NUM_CHUNKS = ROWS_PER_WORKER // CHUNK
NBUF = 8
LOOK = NBUF // 2
assert NUM_CHUNKS % NBUF == 0

_mesh = plsc.VectorSubcoreMesh(
    core_axis_name="c", subcore_axis_name="s",
    num_cores=NUM_CORES, num_subcores=NUM_SUBCORES,
)


@functools.partial(
    pl.kernel,
    out_type=jax.ShapeDtypeStruct((TOTAL, EMBED_DIM), jnp.float32),
    mesh=_mesh,
    scratch_types=[
        pltpu.VMEM((ROWS_PER_WORKER,), jnp.int32),
        [pltpu.VMEM((CHUNK, EMBED_DIM), jnp.float32) for _ in range(NBUF)],
        [pltpu.SemaphoreType.DMA for _ in range(NBUF)],
        [pltpu.SemaphoreType.DMA for _ in range(NBUF)],
    ],
)
def _embedding_gather(idx_hbm, table_hbm, out_hbm, idx_v, bufs, gsems, osems):
    wid = lax.axis_index("s") * NUM_CORES + lax.axis_index("c")
    base = wid * ROWS_PER_WORKER
    pltpu.sync_copy(idx_hbm.at[wid], idx_v)

    def start_gather(j, b):
        chunk_idx = idx_v.at[pl.ds(j * CHUNK, CHUNK)]
        pltpu.async_copy(table_hbm.at[chunk_idx], bufs[b], gsems[b])

    for j in range(LOOK):
        start_gather(j, j % NBUF)

    @pl.loop(0, NUM_CHUNKS, step=NBUF)
    def _outer(j0):
        for b in range(NBUF):
            j = j0 + b
            bn = (b + LOOK) % NBUF
            pltpu.make_async_copy(table_hbm.at[idx_v.at[pl.ds(0, CHUNK)]],
                                  bufs[b], gsems[b]).wait()
            pltpu.async_copy(
                bufs[b], out_hbm.at[pl.ds(base + j * CHUNK, CHUNK)], osems[b])

            @pl.when(j >= NBUF - LOOK)
            def _wait_prev_out():
                pltpu.make_async_copy(
                    bufs[bn], out_hbm.at[pl.ds(base, CHUNK)], osems[bn]).wait()

            @pl.when(j + LOOK < NUM_CHUNKS)
            def _next_gather():
                start_gather(j + LOOK, bn)

    for j in range(NUM_CHUNKS - (NBUF - LOOK), NUM_CHUNKS):
        pltpu.make_async_copy(
            bufs[j % NBUF], out_hbm.at[pl.ds(base, CHUNK)], osems[j % NBUF]).wait()


def kernel(news, table):
    idx = news.T.reshape(NUM_WORKERS, ROWS_PER_WORKER)
    out = _embedding_gather(idx, table)
    return out.reshape(SEQ, BATCH, EMBED_DIM).transpose(1, 0, 2)
